# baseline (speedup 1.0000x reference)
import functools

import jax
import jax.numpy as jnp
from jax import lax
from jax.experimental import pallas as pl
from jax.experimental.pallas import tpu as pltpu
from jax.experimental.pallas import tpu_sc as plsc

D = 32
K = 8192
T = 16384
TB = 256
COMMIT = 0.25

NUM_WORKERS = 32
ROWS_PER_WORKER = T // NUM_WORKERS
IDX_CHUNK = 128


def _argmin_body(x_ref, xsq_ref, et_ref, ema_ref, idx_ref, loss_ref):
    i = pl.program_id(0)
    x = x_ref[...]
    xsq = xsq_ref[...]
    et = et_ref[...]
    ema = ema_ref[...]
    esq = jnp.sum(et * et, axis=0, keepdims=True)
    s = jnp.sqrt(ema)
    mm = lax.dot_general(
        x, et,
        dimension_numbers=(((1,), (0,)), ((), ())),
        preferred_element_type=jnp.float32)
    t2 = (xsq + esq) - 2.0 * mm
    ds = t2 * s
    idx = jnp.argmin(ds, axis=1).astype(jnp.int32)[:, None]
    idx_ref[...] = idx
    iota = lax.broadcasted_iota(jnp.int32, ds.shape, 1)
    part = jnp.sum(jnp.where(iota == idx, t2, 0.0))

    @pl.when(i == 0)
    def _init():
        loss_ref[...] = jnp.zeros_like(loss_ref)

    scale = jnp.where(i == pl.num_programs(0) - 1, COMMIT / (T * D), 1.0)
    loss_ref[...] = (loss_ref[...] + part) * scale


def _tc_argmin(x_flat, xsq_col, e_t, ema_row):
    return pl.pallas_call(
        _argmin_body,
        grid=(T // TB,),
        in_specs=[
            pl.BlockSpec((TB, D), lambda i: (i, 0)),
            pl.BlockSpec((TB, 1), lambda i: (i, 0)),
            pl.BlockSpec((D, K), lambda i: (0, 0)),
            pl.BlockSpec((1, K), lambda i: (0, 0)),
        ],
        out_specs=[
            pl.BlockSpec((TB, 1), lambda i: (i, 0)),
            pl.BlockSpec((1, 1), lambda i: (0, 0)),
        ],
        out_shape=[
            jax.ShapeDtypeStruct((T, 1), jnp.int32),
            jax.ShapeDtypeStruct((1, 1), jnp.float32),
        ],
    )(x_flat, xsq_col, e_t, ema_row)


def _sc_gather_build():
    mesh = plsc.VectorSubcoreMesh(core_axis_name="c", subcore_axis_name="s")

    @functools.partial(
        pl.kernel,
        out_type=jax.ShapeDtypeStruct((T, D), jnp.float32),
        mesh=mesh,
        scratch_types=[
            pltpu.VMEM((ROWS_PER_WORKER,), jnp.int32),
            pltpu.VMEM((ROWS_PER_WORKER, D), jnp.float32),
            pltpu.SemaphoreType.DMA,
        ],
        compiler_params=pltpu.CompilerParams(use_tc_tiling_on_sc=False),
    )
    def gather(table_hbm, idx_hbm, out_hbm, idx_v, rows_v, sem):
        wid = lax.axis_index("s") * 2 + lax.axis_index("c")
        base = wid * ROWS_PER_WORKER
        pltpu.sync_copy(idx_hbm.at[pl.ds(base, ROWS_PER_WORKER)], idx_v)
        for j in range(ROWS_PER_WORKER // IDX_CHUNK):
            pltpu.async_copy(
                table_hbm.at[idx_v.at[pl.ds(j * IDX_CHUNK, IDX_CHUNK)]],
                rows_v.at[pl.ds(j * IDX_CHUNK, IDX_CHUNK)],
                sem,
            ).wait()
        pltpu.sync_copy(rows_v, out_hbm.at[pl.ds(base, ROWS_PER_WORKER)])

    return gather


_sc_gather = _sc_gather_build()


def kernel(inputs, embedding_weight, ema_cluster_size):
    input_shape = inputs.shape
    x_flat = inputs.reshape(T, D)
    xsq_col = jnp.sum(x_flat ** 2, axis=1, keepdims=True)
    e_t = embedding_weight.T
    ema_row = ema_cluster_size.reshape(1, K)
    idx2d, loss2d = _tc_argmin(x_flat, xsq_col, e_t, ema_row)
    quantized = _sc_gather(embedding_weight, idx2d.reshape(T))
    quantized_st = quantized.reshape(input_shape)
    return (quantized_st, loss2d[0, 0], idx2d)
